# baseline (speedup 1.0000x reference)
import functools

import jax
import jax.numpy as jnp
from jax import lax
from jax.experimental import pallas as pl
from jax.experimental.pallas import tpu as pltpu
from jax.experimental.pallas import tpu_sc as plsc

_NUM_CLASSES = 1000
_P = 10
_D = 512
_B = 4096
_M = 0.99

_CPAD = 1024
_NC = 2
_NS = 16
_NRB = 4
_CH = _CPAD // 2
_RB = _B // _NRB
_CBW = 128
_CK = 128
_NCK = _RB // _CK


def _sc_body(feat_h, lab_h, part_h, lab_v, acc_v, bufa, bufb, sema, semb):
    c = lax.axis_index("c")
    s = lax.axis_index("s")
    rb = s % 4
    cbh = (s // 4) % 2
    ch = s // 8
    cb = c * 2 + cbh
    lo = ch * _CH

    row0 = pl.multiple_of(rb * _RB, _CK)
    col0 = pl.multiple_of(cb * _CBW, _CBW)

    zvec = jnp.zeros((16,), jnp.float32)

    def zero_row(i, _):
        for j in range(_CBW // 16):
            acc_v[i, pl.ds(j * 16, 16)] = zvec
        return 0

    lax.fori_loop(0, _CH, zero_row, 0, unroll=8)

    pltpu.sync_copy(lab_h.at[pl.ds(row0, _RB)], lab_v)

    bufs = (bufa, bufb)
    sems = (sema, semb)
    cps = [None, None]
    cps[0] = pltpu.async_copy(
        feat_h.at[pl.ds(row0, _CK), pl.ds(col0, _CBW)], bufa, sema)

    for k in range(_NCK):
        b = k % 2
        cps[b].wait()
        if k + 1 < _NCK:
            nb = (k + 1) % 2
            nxt = pl.multiple_of(row0 + (k + 1) * _CK, _CK)
            cps[nb] = pltpu.async_copy(
                feat_h.at[pl.ds(nxt, _CK), pl.ds(col0, _CBW)], bufs[nb],
                sems[nb])
        buf = bufs[b]

        def group(g, _):
            labv = lab_v[pl.ds(k * _CK + g * 16, 16)]
            for u in range(16):
                lab = labv[u]
                ok = (lab >= lo) & (lab < lo + _CH)
                rel = jnp.where(ok, lab - lo, 0)
                r = g * 16 + u
                for j in range(_CBW // 16):
                    v = buf[r, pl.ds(j * 16, 16)]
                    v = jnp.where(ok, v, 0.0)
                    plsc.addupdate(acc_v.at[rel, pl.ds(j * 16, 16)], v)
            return 0

        lax.fori_loop(0, _CK // 16, group, 0)

    out_r0 = pl.multiple_of(lo, 8)
    pltpu.sync_copy(acc_v,
                    part_h.at[rb, pl.ds(out_r0, _CH), pl.ds(col0, _CBW)])


@functools.cache
def _sc_segment_sum():
    mesh = plsc.VectorSubcoreMesh(
        core_axis_name="c", subcore_axis_name="s",
        num_cores=_NC, num_subcores=_NS,
    )
    return pl.kernel(
        _sc_body,
        out_type=jax.ShapeDtypeStruct((_NRB, _CPAD, _D), jnp.float32),
        mesh=mesh,
        scratch_types=[
            pltpu.VMEM((_RB,), jnp.int32),
            pltpu.VMEM((_CH, _CBW), jnp.float32),
            pltpu.VMEM((_CK, _CBW), jnp.float32),
            pltpu.VMEM((_CK, _CBW), jnp.float32),
            pltpu.SemaphoreType.DMA,
            pltpu.SemaphoreType.DMA,
        ],
    )


_CB = 200


def _tc_blend_body(lab_ref, protos_ref, part_ref, out_ref):
    i = pl.program_id(0)
    cids = i * _CB + lax.broadcasted_iota(jnp.int32, (_CB, 1, 1), 0)
    eq = (lab_ref[...][None, :, :] == cids).astype(jnp.float32)
    cnt = jnp.sum(jnp.sum(eq, axis=2), axis=1).reshape(_CB, 1)
    sums = (part_ref[0] + part_ref[1]) + (part_ref[2] + part_ref[3])
    present = cnt > 0.5
    coef = jnp.where(present, (1.0 - _M) / jnp.maximum(cnt, 1.0), 0.0)
    presentf = jnp.where(present, 1.0, 0.0)
    rows_class = lax.broadcasted_iota(jnp.int32, (_CB * _P, 1), 0) // _P
    onehot = (rows_class == lax.broadcasted_iota(
        jnp.int32, (1, _CB), 1)).astype(jnp.float32)
    addv = jax.lax.dot(onehot, coef * sums,
                       preferred_element_type=jnp.float32)
    pres_row = jax.lax.dot(onehot, presentf,
                           preferred_element_type=jnp.float32)
    fac_row = 1.0 - (1.0 - _M) * pres_row
    out_ref[...] = fac_row * protos_ref[...] + addv


def _tc_blend(lab2d, protos, part):
    return pl.pallas_call(
        _tc_blend_body,
        grid=(_NUM_CLASSES // _CB,),
        in_specs=[
            pl.BlockSpec((_B // 128, 128), lambda i: (0, 0)),
            pl.BlockSpec((_CB * _P, _D), lambda i: (i, 0)),
            pl.BlockSpec((_NRB, _CB, _D), lambda i: (0, i, 0)),
        ],
        out_specs=pl.BlockSpec((_CB * _P, _D), lambda i: (i, 0)),
        out_shape=jax.ShapeDtypeStruct((_NUM_CLASSES * _P, _D), jnp.float32),
    )(lab2d, protos, part)


def kernel(features, labels, prototypes):
    part = _sc_segment_sum()(features, labels)
    lab2d = labels.reshape(_B // 128, 128)
    return _tc_blend(lab2d, prototypes, part)


# speedup vs baseline: 1.0102x; 1.0102x over previous
import functools

import jax
import jax.numpy as jnp
from jax import lax
from jax.experimental import pallas as pl
from jax.experimental.pallas import tpu as pltpu
from jax.experimental.pallas import tpu_sc as plsc

_NUM_CLASSES = 1000
_P = 10
_D = 512
_B = 4096
_M = 0.99

_CPAD = 1024
_NC = 2
_NS = 16
_NRB = 4
_CH = _CPAD // 2
_RB = _B // _NRB
_CBW = 128
_CK = 128
_NCK = _RB // _CK


def _sc_body(feat_h, lab_h, part_h, lab_v, acc_v, bufa, bufb, sema, semb):
    c = lax.axis_index("c")
    s = lax.axis_index("s")
    rb = s % 4
    cbh = (s // 4) % 2
    ch = s // 8
    cb = c * 2 + cbh
    lo = ch * _CH

    row0 = pl.multiple_of(rb * _RB, _CK)
    col0 = pl.multiple_of(cb * _CBW, _CBW)

    zvec = jnp.zeros((16,), jnp.float32)

    def zero_row(i, _):
        for j in range(_CBW // 16):
            acc_v[i, pl.ds(j * 16, 16)] = zvec
        return 0

    lax.fori_loop(0, _CH, zero_row, 0, unroll=8)

    pltpu.sync_copy(lab_h.at[pl.ds(row0, _RB)], lab_v)

    bufs = (bufa, bufb)
    sems = (sema, semb)
    cps = [None, None]
    cps[0] = pltpu.async_copy(
        feat_h.at[pl.ds(row0, _CK), pl.ds(col0, _CBW)], bufa, sema)

    for k in range(_NCK):
        b = k % 2
        cps[b].wait()
        if k + 1 < _NCK:
            nb = (k + 1) % 2
            nxt = pl.multiple_of(row0 + (k + 1) * _CK, _CK)
            cps[nb] = pltpu.async_copy(
                feat_h.at[pl.ds(nxt, _CK), pl.ds(col0, _CBW)], bufs[nb],
                sems[nb])
        buf = bufs[b]

        def group(g, _):
            labv = lab_v[pl.ds(k * _CK + g * 16, 16)]
            for u in range(16):
                lab = labv[u]
                rel = lab - lo

                @pl.when((lab >= lo) & (lab < lo + _CH))
                def _():
                    r = g * 16 + u
                    for j in range(_CBW // 16):
                        plsc.addupdate(acc_v.at[rel, pl.ds(j * 16, 16)],
                                       buf[r, pl.ds(j * 16, 16)])
            return 0

        lax.fori_loop(0, _CK // 16, group, 0, unroll=2)

    out_r0 = pl.multiple_of(lo, 8)
    pltpu.sync_copy(acc_v,
                    part_h.at[rb, pl.ds(out_r0, _CH), pl.ds(col0, _CBW)])


@functools.cache
def _sc_segment_sum():
    mesh = plsc.VectorSubcoreMesh(
        core_axis_name="c", subcore_axis_name="s",
        num_cores=_NC, num_subcores=_NS,
    )
    return pl.kernel(
        _sc_body,
        out_type=jax.ShapeDtypeStruct((_NRB, _CPAD, _D), jnp.float32),
        mesh=mesh,
        scratch_types=[
            pltpu.VMEM((_RB,), jnp.int32),
            pltpu.VMEM((_CH, _CBW), jnp.float32),
            pltpu.VMEM((_CK, _CBW), jnp.float32),
            pltpu.VMEM((_CK, _CBW), jnp.float32),
            pltpu.SemaphoreType.DMA,
            pltpu.SemaphoreType.DMA,
        ],
    )


_CB = 200


def _tc_blend_body(lab_ref, protos_ref, part_ref, out_ref):
    i = pl.program_id(0)
    cids = i * _CB + lax.broadcasted_iota(jnp.int32, (_CB, 1, 1), 0)
    eq = (lab_ref[...][None, :, :] == cids).astype(jnp.float32)
    cnt = jnp.sum(jnp.sum(eq, axis=2), axis=1).reshape(_CB, 1)
    sums = (part_ref[0] + part_ref[1]) + (part_ref[2] + part_ref[3])
    present = cnt > 0.5
    coef = jnp.where(present, (1.0 - _M) / jnp.maximum(cnt, 1.0), 0.0)
    presentf = jnp.where(present, 1.0, 0.0)
    rows_class = lax.broadcasted_iota(jnp.int32, (_CB * _P, 1), 0) // _P
    onehot = (rows_class == lax.broadcasted_iota(
        jnp.int32, (1, _CB), 1)).astype(jnp.float32)
    addv = jax.lax.dot(onehot, coef * sums,
                       preferred_element_type=jnp.float32)
    pres_row = jax.lax.dot(onehot, presentf,
                           preferred_element_type=jnp.float32)
    fac_row = 1.0 - (1.0 - _M) * pres_row
    out_ref[...] = fac_row * protos_ref[...] + addv


def _tc_blend(lab2d, protos, part):
    return pl.pallas_call(
        _tc_blend_body,
        grid=(_NUM_CLASSES // _CB,),
        in_specs=[
            pl.BlockSpec((_B // 128, 128), lambda i: (0, 0)),
            pl.BlockSpec((_CB * _P, _D), lambda i: (i, 0)),
            pl.BlockSpec((_NRB, _CB, _D), lambda i: (0, i, 0)),
        ],
        out_specs=pl.BlockSpec((_CB * _P, _D), lambda i: (i, 0)),
        out_shape=jax.ShapeDtypeStruct((_NUM_CLASSES * _P, _D), jnp.float32),
    )(lab2d, protos, part)


def kernel(features, labels, prototypes):
    part = _sc_segment_sum()(features, labels)
    lab2d = labels.reshape(_B // 128, 128)
    return _tc_blend(lab2d, prototypes, part)


# speedup vs baseline: 1.0785x; 1.0676x over previous
import functools

import jax
import jax.numpy as jnp
from jax import lax
from jax.experimental import pallas as pl
from jax.experimental.pallas import tpu as pltpu
from jax.experimental.pallas import tpu_sc as plsc

_NUM_CLASSES = 1000
_P = 10
_D = 512
_B = 4096
_M = 0.99

_CPAD = 1024
_NC = 2
_NS = 16
_NRB = 4
_CH = _CPAD // 2
_RB = _B // _NRB
_CBW = 128
_CK = 128
_NCK = _RB // _CK


def _sc_body(feat_h, lab_h, part_h, lab_v, acc_v, bufa, bufb, sema, semb):
    c = lax.axis_index("c")
    s = lax.axis_index("s")
    rb = s % 4
    cbh = (s // 4) % 2
    ch = s // 8
    cb = c * 2 + cbh
    lo = ch * _CH

    row0 = pl.multiple_of(rb * _RB, _CK)
    col0 = pl.multiple_of(cb * _CBW, _CBW)

    zvec = jnp.zeros((16,), jnp.float32)

    def zero_row(i, _):
        for j in range(_CBW // 16):
            acc_v[i, pl.ds(j * 16, 16)] = zvec
        return 0

    lax.fori_loop(0, _CH, zero_row, 0, unroll=8)

    pltpu.sync_copy(lab_h.at[pl.ds(row0, _RB)], lab_v)

    bufs = (bufa, bufb)
    sems = (sema, semb)
    cps = [None, None]
    cps[0] = pltpu.async_copy(
        feat_h.at[pl.ds(row0, _CK), pl.ds(col0, _CBW)], bufa, sema)

    for k in range(_NCK):
        b = k % 2
        cps[b].wait()
        if k + 1 < _NCK:
            nb = (k + 1) % 2
            nxt = pl.multiple_of(row0 + (k + 1) * _CK, _CK)
            cps[nb] = pltpu.async_copy(
                feat_h.at[pl.ds(nxt, _CK), pl.ds(col0, _CBW)], bufs[nb],
                sems[nb])
        buf = bufs[b]

        def group(g, _):
            labv = lab_v[pl.ds(k * _CK + g * 16, 16)]
            for u in range(16):
                lab = labv[u]
                rel = lab - lo

                @pl.when((lab >= lo) & (lab < lo + _CH))
                def _():
                    r = g * 16 + u
                    for j in range(_CBW // 16):
                        plsc.addupdate(acc_v.at[rel, pl.ds(j * 16, 16)],
                                       buf[r, pl.ds(j * 16, 16)])
            return 0

        lax.fori_loop(0, _CK // 16, group, 0)

    out_r0 = pl.multiple_of(lo, 8)
    pltpu.sync_copy(acc_v,
                    part_h.at[rb, pl.ds(out_r0, _CH), pl.ds(col0, _CBW)])


@functools.cache
def _sc_segment_sum():
    mesh = plsc.VectorSubcoreMesh(
        core_axis_name="c", subcore_axis_name="s",
        num_cores=_NC, num_subcores=_NS,
    )
    return pl.kernel(
        _sc_body,
        out_type=jax.ShapeDtypeStruct((_NRB, _CPAD, _D), jnp.float32),
        mesh=mesh,
        scratch_types=[
            pltpu.VMEM((_RB,), jnp.int32),
            pltpu.VMEM((_CH, _CBW), jnp.float32),
            pltpu.VMEM((_CK, _CBW), jnp.float32),
            pltpu.VMEM((_CK, _CBW), jnp.float32),
            pltpu.SemaphoreType.DMA,
            pltpu.SemaphoreType.DMA,
        ],
    )


_CB = 200


def _tc_blend_body(lab_ref, protos_ref, part_ref, out_ref):
    i = pl.program_id(0)
    cids = i * _CB + lax.broadcasted_iota(jnp.int32, (_CB, 1, 1), 0)
    eq = (lab_ref[...][None, :, :] == cids).astype(jnp.float32)
    cnt = jnp.sum(jnp.sum(eq, axis=2), axis=1).reshape(_CB, 1)
    sums = (part_ref[0] + part_ref[1]) + (part_ref[2] + part_ref[3])
    present = cnt > 0.5
    coef = jnp.where(present, (1.0 - _M) / jnp.maximum(cnt, 1.0), 0.0)
    presentf = jnp.where(present, 1.0, 0.0)
    rows_class = lax.broadcasted_iota(jnp.int32, (_CB * _P, 1), 0) // _P
    onehot = (rows_class == lax.broadcasted_iota(
        jnp.int32, (1, _CB), 1)).astype(jnp.float32)
    oh16 = onehot.astype(jnp.bfloat16)
    addv = jax.lax.dot(oh16, (coef * sums).astype(jnp.bfloat16),
                       preferred_element_type=jnp.float32)
    pres_row = jax.lax.dot(oh16, presentf.astype(jnp.bfloat16),
                           preferred_element_type=jnp.float32)
    fac_row = 1.0 - (1.0 - _M) * pres_row
    out_ref[...] = fac_row * protos_ref[...] + addv


def _tc_blend(lab2d, protos, part):
    return pl.pallas_call(
        _tc_blend_body,
        grid=(_NUM_CLASSES // _CB,),
        in_specs=[
            pl.BlockSpec((_B // 128, 128), lambda i: (0, 0)),
            pl.BlockSpec((_CB * _P, _D), lambda i: (i, 0)),
            pl.BlockSpec((_NRB, _CB, _D), lambda i: (0, i, 0)),
        ],
        out_specs=pl.BlockSpec((_CB * _P, _D), lambda i: (i, 0)),
        out_shape=jax.ShapeDtypeStruct((_NUM_CLASSES * _P, _D), jnp.float32),
    )(lab2d, protos, part)


def kernel(features, labels, prototypes):
    part = _sc_segment_sum()(features, labels)
    lab2d = labels.reshape(_B // 128, 128)
    return _tc_blend(lab2d, prototypes, part)
